# Initial kernel scaffold; baseline (speedup 1.0000x reference)
import jax
import jax.numpy as jnp
from jax.experimental import pallas as pl


def kernel(input_ids, emb_weight, mix):
    raise NotImplementedError("write your pallas kernel here")


# speedup vs baseline: 2.4155x; 2.4155x over previous
import functools

import jax
import jax.numpy as jnp
from jax import lax
from jax.experimental import pallas as pl
from jax.experimental.pallas import tpu as pltpu
from jax.experimental.pallas import tpu_sc as plsc

VOCAB_SIZE = 100000
MODEL_DIM = 128
CODEBOOK_SIZE = 1000000
NGRAM = 4
MULTIPLIERS = (911382323, 972663749, 97266353, 19260817)

_M1000 = tuple((1000 * m) % CODEBOOK_SIZE for m in MULTIPLIERS)
_M1 = tuple(m % CODEBOOK_SIZE for m in MULTIPLIERS)

_B, _T = 4, 8192
_TOKENS = _B * _T
_NW = 32
_PER_W = _TOKENS // _NW
_CHUNK = 128
_NCHUNK = _PER_W // _CHUNK


def _hash_body(ids_ref, s1_ref, s2_ref, s3_ref, codes_ref):
    codes = ids_ref[...]
    shifted = (s1_ref[...], s2_ref[...], s3_ref[...])
    for offset in range(1, NGRAM):
        a = codes // 1000
        b = codes - a * 1000
        prod = a * _M1000[offset - 1] + b * _M1[offset - 1]
        codes = (prod + shifted[offset - 1] + offset) % CODEBOOK_SIZE
    codes_ref[...] = codes


_hash_call = pl.pallas_call(
    _hash_body,
    out_shape=jax.ShapeDtypeStruct((_B, _T), jnp.int32),
)


def _gather_body(codes_hbm, mix_hbm, table_hbm, out_hbm, idx_v, rows_v, mix_v,
                 sem):
    wid = lax.axis_index("s") * 2 + lax.axis_index("c")
    pltpu.sync_copy(codes_hbm.at[wid], idx_v)
    pltpu.sync_copy(mix_hbm, mix_v)
    m = mix_v[...]
    base = wid * _PER_W
    for c in range(_NCHUNK):
        ci = jnp.asarray(c, jnp.int32)
        pltpu.async_copy(table_hbm.at[idx_v.at[ci]], rows_v, sem).wait()

        def body(r, carry):
            for k in range(MODEL_DIM // 16):
                sl = pl.ds(k * 16, 16)
                rows_v[r, sl] = rows_v[r, sl] * m
            return carry

        lax.fori_loop(0, _CHUNK, body, 0)
        off = (base + c * _CHUNK).astype(jnp.int32)
        pltpu.sync_copy(rows_v, out_hbm.at[pl.ds(off, _CHUNK)])


_gather_call = pl.kernel(
    _gather_body,
    mesh=plsc.VectorSubcoreMesh(core_axis_name="c", subcore_axis_name="s"),
    out_type=jax.ShapeDtypeStruct((_TOKENS, MODEL_DIM), jnp.float32),
    scratch_types=[
        pltpu.VMEM((_NCHUNK, _CHUNK), jnp.int32),
        pltpu.VMEM((_CHUNK, MODEL_DIM), jnp.float32),
        pltpu.VMEM((16,), jnp.float32),
        pltpu.SemaphoreType.DMA,
    ],
)


def kernel(input_ids, emb_weight, mix):
    ids = input_ids.astype(jnp.int32)
    first = ids[:, :1]
    shifts = [
        jnp.concatenate(
            [jnp.broadcast_to(first, (_B, o)), ids[:, :-o]], axis=1)
        for o in range(1, NGRAM)
    ]
    codes = _hash_call(ids, *shifts)
    codes_r = codes.reshape(_NW, _NCHUNK, _CHUNK)
    mix16 = jnp.broadcast_to(mix.astype(jnp.float32), (16,))
    out = _gather_call(codes_r, mix16, emb_weight)
    return out.reshape(_B, _T, MODEL_DIM)


# speedup vs baseline: 2.9470x; 1.2200x over previous
import functools

import jax
import jax.numpy as jnp
from jax import lax
from jax.experimental import pallas as pl
from jax.experimental.pallas import tpu as pltpu
from jax.experimental.pallas import tpu_sc as plsc

VOCAB_SIZE = 100000
MODEL_DIM = 128
CODEBOOK_SIZE = 1000000
NGRAM = 4
MULTIPLIERS = (911382323, 972663749, 97266353, 19260817)

_M1000 = tuple((1000 * m) % CODEBOOK_SIZE for m in MULTIPLIERS)
_M1 = tuple(m % CODEBOOK_SIZE for m in MULTIPLIERS)

_B, _T = 4, 8192
_TOKENS = _B * _T
_NW = 32
_PER_W = _TOKENS // _NW
_CHUNK = 128
_NCHUNK = _PER_W // _CHUNK


def _hash_body(ids_ref, s1_ref, s2_ref, s3_ref, codes_ref):
    codes = ids_ref[...]
    shifted = (s1_ref[...], s2_ref[...], s3_ref[...])
    for offset in range(1, NGRAM):
        a = codes // 1000
        b = codes - a * 1000
        prod = a * _M1000[offset - 1] + b * _M1[offset - 1]
        codes = (prod + shifted[offset - 1] + offset) % CODEBOOK_SIZE
    codes_ref[...] = codes


_hash_call = pl.pallas_call(
    _hash_body,
    out_shape=jax.ShapeDtypeStruct((_B, _T), jnp.int32),
)


_NBUF = 4


def _gather_body(codes_hbm, mix_hbm, table_hbm, out_hbm, idx_v, mix_v,
                 b0, b1, b2, b3, gsem, osem):
    bufs = (b0, b1, b2, b3)
    wid = lax.axis_index("s") * 2 + lax.axis_index("c")
    pltpu.sync_copy(codes_hbm.at[wid], idx_v)
    pltpu.sync_copy(mix_hbm, mix_v)
    m = mix_v[...]
    base = wid * _PER_W

    gds = {}
    wds = {}

    def start_gather(c):
        ci = jnp.asarray(c, jnp.int32)
        nb = c % _NBUF
        gds[c] = pltpu.async_copy(
            table_hbm.at[idx_v.at[ci]], bufs[nb],
            gsem.at[jnp.asarray(nb, jnp.int32)])

    start_gather(0)
    start_gather(1)
    for c in range(_NCHUNK):
        b = c % _NBUF
        if c + 2 < _NCHUNK:
            if c + 2 >= _NBUF:
                wds.pop(c + 2 - _NBUF).wait()
            start_gather(c + 2)
        gds.pop(c).wait()
        buf = bufs[b]

        def body(r, carry):
            for k in range(MODEL_DIM // 16):
                sl = pl.ds(k * 16, 16)
                buf[r, sl] = buf[r, sl] * m
            return carry

        lax.fori_loop(0, _CHUNK, body, 0)
        off = (base + c * _CHUNK).astype(jnp.int32)
        wds[c] = pltpu.async_copy(
            buf, out_hbm.at[pl.ds(off, _CHUNK)],
            osem.at[jnp.asarray(b, jnp.int32)])
    for c in sorted(wds):
        wds[c].wait()


_gather_call = pl.kernel(
    _gather_body,
    mesh=plsc.VectorSubcoreMesh(core_axis_name="c", subcore_axis_name="s"),
    out_type=jax.ShapeDtypeStruct((_TOKENS, MODEL_DIM), jnp.float32),
    scratch_types=[
        pltpu.VMEM((_NCHUNK, _CHUNK), jnp.int32),
        pltpu.VMEM((16,), jnp.float32),
        pltpu.VMEM((_CHUNK, MODEL_DIM), jnp.float32),
        pltpu.VMEM((_CHUNK, MODEL_DIM), jnp.float32),
        pltpu.VMEM((_CHUNK, MODEL_DIM), jnp.float32),
        pltpu.VMEM((_CHUNK, MODEL_DIM), jnp.float32),
        pltpu.SemaphoreType.DMA((_NBUF,)),
        pltpu.SemaphoreType.DMA((_NBUF,)),
    ],
)


def kernel(input_ids, emb_weight, mix):
    ids = input_ids.astype(jnp.int32)
    first = ids[:, :1]
    shifts = [
        jnp.concatenate(
            [jnp.broadcast_to(first, (_B, o)), ids[:, :-o]], axis=1)
        for o in range(1, NGRAM)
    ]
    codes = _hash_call(ids, *shifts)
    codes_r = codes.reshape(_NW, _NCHUNK, _CHUNK)
    mix16 = jnp.broadcast_to(mix.astype(jnp.float32), (16,))
    out = _gather_call(codes_r, mix16, emb_weight)
    return out.reshape(_B, _T, MODEL_DIM)
